# baseline (speedup 1.0000x reference)
import functools

import jax
import jax.numpy as jnp
from jax import lax
from jax.experimental import pallas as pl
from jax.experimental.pallas import tpu as pltpu
from jax.experimental.pallas import tpu_sc as plsc

VOCAB = 1000000
EMBED_DIM = 64
BATCH = 4096
SEQ = 200

NUM_IDX = BATCH * SEQ
IDX_MINOR = 128
IDX_ROWS = NUM_IDX // IDX_MINOR

NC = 2
NS = 16
NW = NC * NS

ROWS_PER_W = IDX_ROWS // NW
CHUNK_ROWS = 4
CHUNK_IDX = CHUNK_ROWS * IDX_MINOR
N_CHUNKS = ROWS_PER_W // CHUNK_ROWS


@functools.partial(
    pl.kernel,
    out_type=jax.ShapeDtypeStruct((NUM_IDX, EMBED_DIM), jnp.float32),
    mesh=plsc.VectorSubcoreMesh(core_axis_name="c", subcore_axis_name="s"),
    scratch_types=[
        pltpu.VMEM((CHUNK_ROWS, IDX_MINOR), jnp.int32),
        pltpu.VMEM((CHUNK_IDX, EMBED_DIM), jnp.float32),
        pltpu.SemaphoreType.DMA,
    ],
    compiler_params=pltpu.CompilerParams(use_tc_tiling_on_sc=False),
)
def _gather_sc(table_hbm, idx_hbm, out_hbm, idx_v, rows_v, gsem):
    wid = lax.axis_index("s") * NC + lax.axis_index("c")
    base = wid * ROWS_PER_W

    def chunk_body(c, carry):
        r0 = base + c * CHUNK_ROWS
        pltpu.sync_copy(idx_hbm.at[pl.ds(r0, CHUNK_ROWS)], idx_v)
        copies = [
            pltpu.async_copy(
                table_hbm.at[idx_v.at[j]],
                rows_v.at[pl.ds(j * IDX_MINOR, IDX_MINOR)],
                gsem,
            )
            for j in range(CHUNK_ROWS)
        ]
        for cp in copies:
            cp.wait()
        pltpu.sync_copy(rows_v, out_hbm.at[pl.ds(r0 * IDX_MINOR, CHUNK_IDX)])
        return carry

    lax.fori_loop(0, N_CHUNKS, chunk_body, 0)


def kernel(input, table):
    idx = input.reshape(IDX_ROWS, IDX_MINOR)
    out = _gather_sc(table, idx)
    return out.reshape(BATCH, SEQ, EMBED_DIM)


# speedup vs baseline: 1.0432x; 1.0432x over previous
import functools

import jax
import jax.numpy as jnp
from jax import lax
from jax.experimental import pallas as pl
from jax.experimental.pallas import tpu as pltpu
from jax.experimental.pallas import tpu_sc as plsc

VOCAB = 1000000
EMBED_DIM = 64
BATCH = 4096
SEQ = 200

NUM_IDX = BATCH * SEQ
IDX_MINOR = 128
IDX_ROWS = NUM_IDX // IDX_MINOR

NC = 2
NS = 16
NW = NC * NS

ROWS_PER_W = IDX_ROWS // NW
ROWS_PER_CHUNK = 2
CHUNK = ROWS_PER_CHUNK * IDX_MINOR
NBUF = 4
N_CHUNKS = ROWS_PER_W // ROWS_PER_CHUNK
N_STEADY = N_CHUNKS // NBUF - 1


@functools.partial(
    pl.kernel,
    out_type=jax.ShapeDtypeStruct((NUM_IDX, EMBED_DIM), jnp.float32),
    mesh=plsc.VectorSubcoreMesh(core_axis_name="c", subcore_axis_name="s"),
    scratch_types=[
        pltpu.VMEM((ROWS_PER_W, IDX_MINOR), jnp.int32),
        pltpu.VMEM((NBUF, CHUNK, EMBED_DIM), jnp.float32),
        [pltpu.SemaphoreType.DMA] * NBUF,
        [pltpu.SemaphoreType.DMA] * NBUF,
    ],
    compiler_params=pltpu.CompilerParams(use_tc_tiling_on_sc=False),
)
def _gather_sc(table_hbm, idx_hbm, out_hbm, idx_all, rows_v, gsems, wsems):
    wid = lax.axis_index("s") * NC + lax.axis_index("c")
    base = wid * ROWS_PER_W

    pltpu.sync_copy(idx_hbm.at[pl.ds(base, ROWS_PER_W)], idx_all)

    def start_gather(c, b):
        for j in range(ROWS_PER_CHUNK):
            pltpu.async_copy(
                table_hbm.at[idx_all.at[c * ROWS_PER_CHUNK + j]],
                rows_v.at[b, pl.ds(j * IDX_MINOR, IDX_MINOR)],
                gsems[b],
            )

    def wait_gather(b):
        pltpu.make_async_copy(
            table_hbm.at[pl.ds(0, CHUNK)], rows_v.at[b], gsems[b]
        ).wait()

    def start_write(c, b):
        out_off = (base + c * ROWS_PER_CHUNK) * IDX_MINOR
        return pltpu.async_copy(
            rows_v.at[b], out_hbm.at[pl.ds(out_off, CHUNK)], wsems[b]
        )

    def wait_write(b):
        pltpu.make_async_copy(
            rows_v.at[b], out_hbm.at[pl.ds(0, CHUNK)], wsems[b]
        ).wait()

    for b in range(NBUF):
        start_gather(b, b)

    def steady(p, carry):
        c0 = p * NBUF
        for b in range(NBUF):
            wait_gather(b)
            start_write(c0 + b, b)
        for b in range(NBUF):
            wait_write(b)
            start_gather(c0 + NBUF + b, b)
        return carry

    lax.fori_loop(0, N_STEADY, steady, 0)

    for b in range(NBUF):
        wait_gather(b)
        start_write(N_CHUNKS - NBUF + b, b)
    for b in range(NBUF):
        wait_write(b)


def kernel(input, table):
    idx = input.reshape(IDX_ROWS, IDX_MINOR)
    out = _gather_sc(table, idx)
    return out.reshape(BATCH, SEQ, EMBED_DIM)
